# baseline (speedup 1.0000x reference)
import functools

import jax
import jax.numpy as jnp
from jax import lax
from jax.experimental import pallas as pl
from jax.experimental.pallas import tpu as pltpu
from jax.experimental.pallas import tpu_sc as plsc

N = 10000
D = 128
E = 320000
HID = 128
DW = 16
AUG = D + DW

NC = 2
NS = 16
NW = NC * NS
EPW = E // NW
ECH = 40
NECH = EPW // ECH
NBUF = 5
NGRP = NECH // NBUF
RCH = 40
NRCH = N // RCH
RK = (NRCH + NS - 1) // NS

BLK = 1000
GRID = N // BLK


def _sc_aggregate(x, ei, z128, z16, ones16):
    mesh = plsc.VectorSubcoreMesh(core_axis_name="c", subcore_axis_name="s")

    @functools.partial(
        pl.kernel,
        out_type=(jax.ShapeDtypeStruct((2 * N, D), jnp.float32),
                  jax.ShapeDtypeStruct((2 * N, DW), jnp.float32)),
        mesh=mesh,
        compiler_params=pltpu.CompilerParams(use_tc_tiling_on_sc=False),
        scratch_types=[
            [pltpu.VMEM((2, ECH), jnp.int32) for _ in range(NBUF)],
            [pltpu.VMEM((2, ECH), jnp.int32) for _ in range(NBUF)],
            [pltpu.VMEM((ECH, D), jnp.float32) for _ in range(NBUF)],
            pltpu.VMEM((ECH, DW), jnp.float32),
            pltpu.VMEM((RCH, D), jnp.float32),
            pltpu.VMEM((RCH, DW), jnp.float32),
            pltpu.VMEM_SHARED((N, D), jnp.float32),
            pltpu.VMEM_SHARED((N, DW), jnp.float32),
            [pltpu.SemaphoreType.DMA for _ in range(NBUF)],
            [pltpu.SemaphoreType.DMA for _ in range(NBUF)],
            [pltpu.SemaphoreType.DMA for _ in range(NBUF)],
            [pltpu.SemaphoreType.DMA for _ in range(NBUF)],
        ],
    )
    def sc_kernel(x_hbm, ei_hbm, z128_hbm, z16_hbm, ones_hbm,
                  outv_hbm, outd_hbm,
                  crA, crB, bufs, onesb, z128b, z16b,
                  acc_sh, dacc_sh, iA, iB, gsem, ssem):
        cid = lax.axis_index("c")
        sid = lax.axis_index("s")
        wid = sid * NC + cid
        ebase = wid * EPW

        def fire_idx(i, crb, sem):
            off = ebase + i * ECH
            pltpu.async_copy(ei_hbm.at[:, pl.ds(off, ECH)], crb, sem)

        def wait_idx(i, crb, sem):
            off = ebase + i * ECH
            pltpu.make_async_copy(ei_hbm.at[:, pl.ds(off, ECH)], crb, sem).wait()

        for b in range(NBUF):
            fire_idx(b, crA[b], iA[b])
            fire_idx(b + NBUF, crB[b], iB[b])
        pltpu.sync_copy(z128_hbm, z128b)
        pltpu.sync_copy(z16_hbm, z16b)
        pltpu.sync_copy(ones_hbm, onesb)
        for b in range(NBUF):
            wait_idx(b, crA[b], iA[b])
            pltpu.async_copy(x_hbm.at[crA[b].at[1]], bufs[b], gsem[b])

        def zero_body(k, _):
            c = sid + k * NS

            @pl.when(c < NRCH)
            def _():
                pltpu.sync_copy(z128b, acc_sh.at[pl.ds(c * RCH, RCH)])
                pltpu.sync_copy(z16b, dacc_sh.at[pl.ds(c * RCH, RCH)])
            return _

        lax.fori_loop(0, RK, zero_body, None)
        plsc.subcore_barrier()

        def half(base, crX, crY, isX, isY):
            for b in range(NBUF):
                pltpu.make_async_copy(x_hbm.at[crX[b].at[1]], bufs[b],
                                      gsem[b]).wait()
                pltpu.async_copy(bufs[b], acc_sh.at[crX[b].at[0]], ssem[b],
                                 add=True)
                pltpu.async_copy(onesb, dacc_sh.at[crX[b].at[0]], ssem[b],
                                 add=True)
            for b in range(NBUF):
                i = base + b
                pltpu.make_async_copy(bufs[b], acc_sh.at[crX[b].at[0]],
                                      ssem[b]).wait()
                pltpu.make_async_copy(onesb, dacc_sh.at[crX[b].at[0]],
                                      ssem[b]).wait()

                @pl.when(i + 2 * NBUF < NECH)
                def _():
                    fire_idx(i + 2 * NBUF, crX[b], isX[b])

                @pl.when(i + NBUF < NECH)
                def _():
                    wait_idx(i + NBUF, crY[b], isY[b])
                    pltpu.async_copy(x_hbm.at[crY[b].at[1]], bufs[b], gsem[b])

        def pair(gg, _):
            base0 = gg * 2 * NBUF
            half(base0, crA, crB, iA, iB)
            half(base0 + NBUF, crB, crA, iB, iA)
            return _

        lax.fori_loop(0, NGRP // 2, pair, None)
        plsc.subcore_barrier()

        def wb_body(k, _):
            c = sid + k * NS

            @pl.when(c < NRCH)
            def _():
                pltpu.sync_copy(acc_sh.at[pl.ds(c * RCH, RCH)], z128b)
                pltpu.sync_copy(z128b,
                                outv_hbm.at[pl.ds(cid * N + c * RCH, RCH)])
                pltpu.sync_copy(dacc_sh.at[pl.ds(c * RCH, RCH)], z16b)
                pltpu.sync_copy(z16b,
                                outd_hbm.at[pl.ds(cid * N + c * RCH, RCH)])
            return _

        lax.fori_loop(0, RK, wb_body, None)

    return sc_kernel(x, ei, z128, z16, ones16)


def _mlp_body(x_ref, v0_ref, v1_ref, pd_ref,
              w1_ref, b1_ref, w2_ref, b2_ref,
              w3_ref, b3_ref, w4_ref, b4_ref, out_ref,
              d0v, d1v, dsem):
    def matt(a, w):
        return lax.dot_general(a, w, (((1,), (1,)), ((), ())),
                               preferred_element_type=jnp.float32)

    i = pl.program_id(0)
    c0 = pltpu.make_async_copy(pd_ref.at[pl.ds(i * BLK, BLK)], d0v, dsem)
    c1 = pltpu.make_async_copy(pd_ref.at[pl.ds(N + i * BLK, BLK)], d1v, dsem)
    c0.start()
    c1.start()
    s = v0_ref[...] + v1_ref[...]
    c0.wait()
    c1.wait()
    deg = jnp.maximum(d0v[:, :1] + d1v[:, :1], 1.0)
    agg = s / deg
    h = jnp.concatenate([x_ref[...], agg], axis=1)
    h = jnp.maximum(matt(h, w1_ref[...]) + b1_ref[...], 0.0)
    h = jnp.maximum(matt(h, w2_ref[...]) + b2_ref[...], 0.0)
    h = jnp.maximum(matt(h, w3_ref[...]) + b3_ref[...], 0.0)
    out_ref[...] = matt(h, w4_ref[...]) + b4_ref[...]


def _tc_mlp(x, pv, pd, w1t, b1, w2t, b2, w3t, b3, w4t, b4):
    def wspec(shape):
        return pl.BlockSpec(shape, lambda i: (0, 0))

    return pl.pallas_call(
        _mlp_body,
        grid=(GRID,),
        in_specs=[
            pl.BlockSpec((BLK, D), lambda i: (i, 0)),
            pl.BlockSpec((BLK, D), lambda i: (i, 0)),
            pl.BlockSpec((BLK, D), lambda i: (i + GRID, 0)),
            pl.BlockSpec(memory_space=pltpu.MemorySpace.HBM),
            wspec((HID, 2 * D)), wspec((1, HID)),
            wspec((HID, HID)), wspec((1, HID)),
            wspec((HID, HID)), wspec((1, HID)),
            wspec((D, HID)), wspec((1, D)),
        ],
        out_specs=pl.BlockSpec((BLK, D), lambda i: (i, 0)),
        out_shape=jax.ShapeDtypeStruct((N, D), jnp.float32),
        scratch_shapes=[
            pltpu.VMEM((BLK, DW), jnp.float32),
            pltpu.VMEM((BLK, DW), jnp.float32),
            pltpu.SemaphoreType.DMA,
        ],
    )(x, pv, pv, pd, w1t, b1, w2t, b2, w3t, b3, w4t, b4)


def kernel(x, edge_index, W1, b1, W2, b2, W3, b3, W4, b4):
    ei = edge_index.astype(jnp.int32)
    z128 = jnp.zeros((RCH, D), jnp.float32)
    z16 = jnp.zeros((RCH, DW), jnp.float32)
    ones16 = jnp.ones((ECH, DW), jnp.float32)

    pv, pd = _sc_aggregate(x, ei, z128, z16, ones16)

    return _tc_mlp(
        x, pv, pd,
        W1, b1.reshape(1, HID),
        W2, b2.reshape(1, HID),
        W3, b3.reshape(1, HID),
        W4, b4.reshape(1, D),
    )


# speedup vs baseline: 1.1276x; 1.1276x over previous
import functools

import jax
import jax.numpy as jnp
from jax import lax
from jax.experimental import pallas as pl
from jax.experimental.pallas import tpu as pltpu
from jax.experimental.pallas import tpu_sc as plsc

N = 10000
D = 128
E = 320000
HID = 128
DW = 16
AUG = D + DW

NC = 2
NS = 16
NW = NC * NS
EPW = E // NW
ECH = 40
NECH = EPW // ECH
NBUF = 5
NGRP = NECH // NBUF
RCH = 40
NRCH = N // RCH
RK = (NRCH + NS - 1) // NS

BLK = 1000
GRID = N // BLK


def _sc_aggregate(x, ei, z128, z16, ones16):
    mesh = plsc.VectorSubcoreMesh(core_axis_name="c", subcore_axis_name="s")

    @functools.partial(
        pl.kernel,
        out_type=(jax.ShapeDtypeStruct((2 * N, D), jnp.float32),
                  jax.ShapeDtypeStruct((2 * N, D), jnp.float32)),
        mesh=mesh,
        compiler_params=pltpu.CompilerParams(use_tc_tiling_on_sc=False),
        scratch_types=[
            [pltpu.VMEM((2, ECH), jnp.int32) for _ in range(NBUF)],
            [pltpu.VMEM((2, ECH), jnp.int32) for _ in range(NBUF)],
            [pltpu.VMEM((ECH, D), jnp.float32) for _ in range(NBUF)],
            pltpu.VMEM((ECH, DW), jnp.float32),
            pltpu.VMEM((RCH, D), jnp.float32),
            pltpu.VMEM((RCH, DW), jnp.float32),
            pltpu.VMEM_SHARED((N, D), jnp.float32),
            pltpu.VMEM_SHARED((N, DW), jnp.float32),
            [pltpu.SemaphoreType.DMA for _ in range(NBUF)],
            [pltpu.SemaphoreType.DMA for _ in range(NBUF)],
            [pltpu.SemaphoreType.DMA for _ in range(NBUF)],
            [pltpu.SemaphoreType.DMA for _ in range(NBUF)],
        ],
    )
    def sc_kernel(x_hbm, ei_hbm, z128_hbm, z16_hbm, ones_hbm,
                  outv_hbm, outd_hbm,
                  crA, crB, bufs, onesb, z128b, z16b,
                  acc_sh, dacc_sh, iA, iB, gsem, ssem):
        cid = lax.axis_index("c")
        sid = lax.axis_index("s")
        wid = sid * NC + cid
        ebase = wid * EPW

        def fire_idx(i, crb, sem):
            off = ebase + i * ECH
            pltpu.async_copy(ei_hbm.at[:, pl.ds(off, ECH)], crb, sem)

        def wait_idx(i, crb, sem):
            off = ebase + i * ECH
            pltpu.make_async_copy(ei_hbm.at[:, pl.ds(off, ECH)], crb, sem).wait()

        for b in range(NBUF):
            fire_idx(b, crA[b], iA[b])
            fire_idx(b + NBUF, crB[b], iB[b])
        pltpu.sync_copy(z128_hbm, z128b)
        pltpu.sync_copy(z16_hbm, z16b)
        pltpu.sync_copy(ones_hbm, onesb)
        for b in range(NBUF):
            wait_idx(b, crA[b], iA[b])
            pltpu.async_copy(x_hbm.at[crA[b].at[1]], bufs[b], gsem[b])

        def zero_body(k, _):
            c = sid + k * NS

            @pl.when(c < NRCH)
            def _():
                pltpu.sync_copy(z128b, acc_sh.at[pl.ds(c * RCH, RCH)])
                pltpu.sync_copy(z16b, dacc_sh.at[pl.ds(c * RCH, RCH)])
            return _

        lax.fori_loop(0, RK, zero_body, None)
        plsc.subcore_barrier()

        def half(base, crX, crY, isX, isY):
            for b in range(NBUF):
                pltpu.make_async_copy(x_hbm.at[crX[b].at[1]], bufs[b],
                                      gsem[b]).wait()
                pltpu.async_copy(bufs[b], acc_sh.at[crX[b].at[0]], ssem[b],
                                 add=True)
                pltpu.async_copy(onesb, dacc_sh.at[crX[b].at[0]], ssem[b],
                                 add=True)
            for b in range(NBUF):
                i = base + b
                pltpu.make_async_copy(bufs[b], acc_sh.at[crX[b].at[0]],
                                      ssem[b]).wait()
                pltpu.make_async_copy(onesb, dacc_sh.at[crX[b].at[0]],
                                      ssem[b]).wait()

                @pl.when(i + 2 * NBUF < NECH)
                def _():
                    fire_idx(i + 2 * NBUF, crX[b], isX[b])

                @pl.when(i + NBUF < NECH)
                def _():
                    wait_idx(i + NBUF, crY[b], isY[b])
                    pltpu.async_copy(x_hbm.at[crY[b].at[1]], bufs[b], gsem[b])

        def pair(gg, _):
            base0 = gg * 2 * NBUF
            half(base0, crA, crB, iA, iB)
            half(base0 + NBUF, crB, crA, iB, iA)
            return _

        lax.fori_loop(0, NGRP // 2, pair, None)
        plsc.subcore_barrier()

        def wb_body(k, _):
            c = sid + k * NS

            @pl.when(c < NRCH)
            def _():
                pltpu.sync_copy(acc_sh.at[pl.ds(c * RCH, RCH)], z128b)
                pltpu.sync_copy(z128b,
                                outv_hbm.at[pl.ds(cid * N + c * RCH, RCH)])
                pltpu.sync_copy(dacc_sh.at[pl.ds(c * RCH, RCH)], z16b)
                pltpu.sync_copy(z16b,
                                outd_hbm.at[pl.ds(cid * N + c * RCH, RCH),
                                            pl.ds(0, DW)])
            return _

        lax.fori_loop(0, RK, wb_body, None)

    return sc_kernel(x, ei, z128, z16, ones16)


def _mlp_body(x_ref, v0_ref, v1_ref, d0_ref, d1_ref,
              w1_ref, b1_ref, w2_ref, b2_ref,
              w3_ref, b3_ref, w4_ref, b4_ref, out_ref):
    def matt(a, w):
        return lax.dot_general(a, w, (((1,), (1,)), ((), ())),
                               preferred_element_type=jnp.float32)

    s = v0_ref[...] + v1_ref[...]
    deg = jnp.maximum(d0_ref[:, :1] + d1_ref[:, :1], 1.0)
    agg = s / deg
    h = jnp.concatenate([x_ref[...], agg], axis=1)
    h = jnp.maximum(matt(h, w1_ref[...]) + b1_ref[...], 0.0)
    h = jnp.maximum(matt(h, w2_ref[...]) + b2_ref[...], 0.0)
    h = jnp.maximum(matt(h, w3_ref[...]) + b3_ref[...], 0.0)
    out_ref[...] = matt(h, w4_ref[...]) + b4_ref[...]


def _tc_mlp(x, pv, pd, w1t, b1, w2t, b2, w3t, b3, w4t, b4):
    def wspec(shape):
        return pl.BlockSpec(shape, lambda i: (0, 0))

    return pl.pallas_call(
        _mlp_body,
        grid=(GRID,),
        in_specs=[
            pl.BlockSpec((BLK, D), lambda i: (i, 0)),
            pl.BlockSpec((BLK, D), lambda i: (i, 0)),
            pl.BlockSpec((BLK, D), lambda i: (i + GRID, 0)),
            pl.BlockSpec((BLK, D), lambda i: (i, 0)),
            pl.BlockSpec((BLK, D), lambda i: (i + GRID, 0)),
            wspec((HID, 2 * D)), wspec((1, HID)),
            wspec((HID, HID)), wspec((1, HID)),
            wspec((HID, HID)), wspec((1, HID)),
            wspec((D, HID)), wspec((1, D)),
        ],
        out_specs=pl.BlockSpec((BLK, D), lambda i: (i, 0)),
        out_shape=jax.ShapeDtypeStruct((N, D), jnp.float32),
    )(x, pv, pv, pd, pd, w1t, b1, w2t, b2, w3t, b3, w4t, b4)


def kernel(x, edge_index, W1, b1, W2, b2, W3, b3, W4, b4):
    ei = edge_index.astype(jnp.int32)
    z128 = jnp.zeros((RCH, D), jnp.float32)
    z16 = jnp.zeros((RCH, DW), jnp.float32)
    ones16 = jnp.ones((ECH, DW), jnp.float32)

    pv, pd = _sc_aggregate(x, ei, z128, z16, ones16)

    return _tc_mlp(
        x, pv, pd,
        W1, b1.reshape(1, HID),
        W2, b2.reshape(1, HID),
        W3, b3.reshape(1, HID),
        W4, b4.reshape(1, D),
    )


# speedup vs baseline: 1.1406x; 1.0115x over previous
import functools

import jax
import jax.numpy as jnp
from jax import lax
from jax.experimental import pallas as pl
from jax.experimental.pallas import tpu as pltpu
from jax.experimental.pallas import tpu_sc as plsc

N = 10000
D = 128
E = 320000
HID = 128
DW = 16
AUG = D + DW

NC = 2
NS = 16
NW = NC * NS
EPW = E // NW
ECH = 40
NECH = EPW // ECH
NBUF = 5
NGRP = NECH // NBUF
RCH = 40
NRCH = N // RCH
RK = (NRCH + NS - 1) // NS

BLK = 2000
GRID = N // BLK


def _sc_aggregate(x, ei, z128, z16, ones16):
    mesh = plsc.VectorSubcoreMesh(core_axis_name="c", subcore_axis_name="s")

    @functools.partial(
        pl.kernel,
        out_type=(jax.ShapeDtypeStruct((2 * N, D), jnp.float32),
                  jax.ShapeDtypeStruct((2 * N, D), jnp.float32)),
        mesh=mesh,
        compiler_params=pltpu.CompilerParams(use_tc_tiling_on_sc=False),
        scratch_types=[
            [pltpu.VMEM((2, ECH), jnp.int32) for _ in range(NBUF)],
            [pltpu.VMEM((2, ECH), jnp.int32) for _ in range(NBUF)],
            [pltpu.VMEM((ECH, D), jnp.float32) for _ in range(NBUF)],
            pltpu.VMEM((ECH, DW), jnp.float32),
            pltpu.VMEM((RCH, D), jnp.float32),
            pltpu.VMEM((RCH, DW), jnp.float32),
            pltpu.VMEM_SHARED((N, D), jnp.float32),
            pltpu.VMEM_SHARED((N, DW), jnp.float32),
            [pltpu.SemaphoreType.DMA for _ in range(NBUF)],
            [pltpu.SemaphoreType.DMA for _ in range(NBUF)],
            [pltpu.SemaphoreType.DMA for _ in range(NBUF)],
            [pltpu.SemaphoreType.DMA for _ in range(NBUF)],
        ],
    )
    def sc_kernel(x_hbm, ei_hbm, z128_hbm, z16_hbm, ones_hbm,
                  outv_hbm, outd_hbm,
                  crA, crB, bufs, onesb, z128b, z16b,
                  acc_sh, dacc_sh, iA, iB, gsem, ssem):
        cid = lax.axis_index("c")
        sid = lax.axis_index("s")
        wid = sid * NC + cid
        ebase = wid * EPW

        def fire_idx(i, crb, sem):
            off = ebase + i * ECH
            pltpu.async_copy(ei_hbm.at[:, pl.ds(off, ECH)], crb, sem)

        def wait_idx(i, crb, sem):
            off = ebase + i * ECH
            pltpu.make_async_copy(ei_hbm.at[:, pl.ds(off, ECH)], crb, sem).wait()

        for b in range(NBUF):
            fire_idx(b, crA[b], iA[b])
            fire_idx(b + NBUF, crB[b], iB[b])
        pltpu.sync_copy(z128_hbm, z128b)
        pltpu.sync_copy(z16_hbm, z16b)
        pltpu.sync_copy(ones_hbm, onesb)
        for b in range(NBUF):
            wait_idx(b, crA[b], iA[b])
            pltpu.async_copy(x_hbm.at[crA[b].at[1]], bufs[b], gsem[b])

        def zero_body(k, _):
            c = sid + k * NS

            @pl.when(c < NRCH)
            def _():
                pltpu.sync_copy(z128b, acc_sh.at[pl.ds(c * RCH, RCH)])
                pltpu.sync_copy(z16b, dacc_sh.at[pl.ds(c * RCH, RCH)])
            return _

        lax.fori_loop(0, RK, zero_body, None)
        plsc.subcore_barrier()

        def half(base, crX, crY, isX, isY):
            for b in range(NBUF):
                pltpu.make_async_copy(x_hbm.at[crX[b].at[1]], bufs[b],
                                      gsem[b]).wait()
                pltpu.async_copy(bufs[b], acc_sh.at[crX[b].at[0]], ssem[b],
                                 add=True)
                pltpu.async_copy(onesb, dacc_sh.at[crX[b].at[0]], ssem[b],
                                 add=True)
            for b in range(NBUF):
                i = base + b
                pltpu.make_async_copy(bufs[b], acc_sh.at[crX[b].at[0]],
                                      ssem[b]).wait()
                pltpu.make_async_copy(onesb, dacc_sh.at[crX[b].at[0]],
                                      ssem[b]).wait()

                @pl.when(i + 2 * NBUF < NECH)
                def _():
                    fire_idx(i + 2 * NBUF, crX[b], isX[b])

                @pl.when(i + NBUF < NECH)
                def _():
                    wait_idx(i + NBUF, crY[b], isY[b])
                    pltpu.async_copy(x_hbm.at[crY[b].at[1]], bufs[b], gsem[b])

        def pair(gg, _):
            base0 = gg * 2 * NBUF
            half(base0, crA, crB, iA, iB)
            half(base0 + NBUF, crB, crA, iB, iA)
            return _

        lax.fori_loop(0, NGRP // 2, pair, None)
        plsc.subcore_barrier()

        def wb_body(k, _):
            c = sid + k * NS

            @pl.when(c < NRCH)
            def _():
                pltpu.sync_copy(acc_sh.at[pl.ds(c * RCH, RCH)], z128b)
                pltpu.sync_copy(z128b,
                                outv_hbm.at[pl.ds(cid * N + c * RCH, RCH)])
                pltpu.sync_copy(dacc_sh.at[pl.ds(c * RCH, RCH)], z16b)
                pltpu.sync_copy(z16b,
                                outd_hbm.at[pl.ds(cid * N + c * RCH, RCH),
                                            pl.ds(0, DW)])
            return _

        lax.fori_loop(0, RK, wb_body, None)

    return sc_kernel(x, ei, z128, z16, ones16)


def _mlp_body(x_ref, v0_ref, v1_ref, d0_ref, d1_ref,
              w1_ref, b1_ref, w2_ref, b2_ref,
              w3_ref, b3_ref, w4_ref, b4_ref, out_ref):
    def matt(a, w):
        return lax.dot_general(a, w, (((1,), (1,)), ((), ())),
                               preferred_element_type=jnp.float32)

    s = v0_ref[...] + v1_ref[...]
    deg = jnp.maximum(d0_ref[:, :1] + d1_ref[:, :1], 1.0)
    agg = s / deg
    h = jnp.concatenate([x_ref[...], agg], axis=1)
    h = jnp.maximum(matt(h, w1_ref[...]) + b1_ref[...], 0.0)
    h = jnp.maximum(matt(h, w2_ref[...]) + b2_ref[...], 0.0)
    h = jnp.maximum(matt(h, w3_ref[...]) + b3_ref[...], 0.0)
    out_ref[...] = matt(h, w4_ref[...]) + b4_ref[...]


def _tc_mlp(x, pv, pd, w1t, b1, w2t, b2, w3t, b3, w4t, b4):
    def wspec(shape):
        return pl.BlockSpec(shape, lambda i: (0, 0))

    return pl.pallas_call(
        _mlp_body,
        grid=(GRID,),
        in_specs=[
            pl.BlockSpec((BLK, D), lambda i: (i, 0)),
            pl.BlockSpec((BLK, D), lambda i: (i, 0)),
            pl.BlockSpec((BLK, D), lambda i: (i + GRID, 0)),
            pl.BlockSpec((BLK, D), lambda i: (i, 0)),
            pl.BlockSpec((BLK, D), lambda i: (i + GRID, 0)),
            wspec((HID, 2 * D)), wspec((1, HID)),
            wspec((HID, HID)), wspec((1, HID)),
            wspec((HID, HID)), wspec((1, HID)),
            wspec((D, HID)), wspec((1, D)),
        ],
        out_specs=pl.BlockSpec((BLK, D), lambda i: (i, 0)),
        out_shape=jax.ShapeDtypeStruct((N, D), jnp.float32),
    )(x, pv, pv, pd, pd, w1t, b1, w2t, b2, w3t, b3, w4t, b4)


def kernel(x, edge_index, W1, b1, W2, b2, W3, b3, W4, b4):
    ei = edge_index.astype(jnp.int32)
    z128 = jnp.zeros((RCH, D), jnp.float32)
    z16 = jnp.zeros((RCH, DW), jnp.float32)
    ones16 = jnp.ones((ECH, DW), jnp.float32)

    pv, pd = _sc_aggregate(x, ei, z128, z16, ones16)

    return _tc_mlp(
        x, pv, pd,
        W1, b1.reshape(1, HID),
        W2, b2.reshape(1, HID),
        W3, b3.reshape(1, HID),
        W4, b4.reshape(1, D),
    )


# speedup vs baseline: 1.1431x; 1.0021x over previous
import functools

import jax
import jax.numpy as jnp
from jax import lax
from jax.experimental import pallas as pl
from jax.experimental.pallas import tpu as pltpu
from jax.experimental.pallas import tpu_sc as plsc

N = 10000
D = 128
E = 320000
HID = 128
DW = 16
AUG = D + DW

NC = 2
NS = 16
NW = NC * NS
EPW = E // NW
ECH = 40
NECH = EPW // ECH
NBUF = 5
NGRP = NECH // NBUF
RCH = 40
NRCH = N // RCH
RK = (NRCH + NS - 1) // NS

BLK = 2000
GRID = N // BLK


def _sc_aggregate(x, ei, z128, z16, ones16):
    mesh = plsc.VectorSubcoreMesh(core_axis_name="c", subcore_axis_name="s")

    @functools.partial(
        pl.kernel,
        out_type=(jax.ShapeDtypeStruct((2 * N, D), jnp.float32),
                  jax.ShapeDtypeStruct((2 * N, D), jnp.float32)),
        mesh=mesh,
        compiler_params=pltpu.CompilerParams(use_tc_tiling_on_sc=False),
        scratch_types=[
            [pltpu.VMEM((2, ECH), jnp.int32) for _ in range(NBUF)],
            [pltpu.VMEM((2, ECH), jnp.int32) for _ in range(NBUF)],
            [pltpu.VMEM((ECH, D), jnp.float32) for _ in range(NBUF)],
            pltpu.VMEM((ECH, DW), jnp.float32),
            pltpu.VMEM((RCH, D), jnp.float32),
            pltpu.VMEM((RCH, DW), jnp.float32),
            pltpu.VMEM_SHARED((N, D), jnp.float32),
            pltpu.VMEM_SHARED((N, DW), jnp.float32),
            [pltpu.SemaphoreType.DMA for _ in range(NBUF)],
            [pltpu.SemaphoreType.DMA for _ in range(NBUF)],
            [pltpu.SemaphoreType.DMA for _ in range(NBUF)],
            [pltpu.SemaphoreType.DMA for _ in range(NBUF)],
        ],
    )
    def sc_kernel(x_hbm, ei_hbm, z128_hbm, z16_hbm, ones_hbm,
                  outv_hbm, outd_hbm,
                  crA, crB, bufs, onesb, z128b, z16b,
                  acc_sh, dacc_sh, iA, iB, gsem, ssem):
        cid = lax.axis_index("c")
        sid = lax.axis_index("s")
        wid = sid * NC + cid
        ebase = wid * EPW

        def fire_idx(i, crb, sem):
            off = ebase + i * ECH
            pltpu.async_copy(ei_hbm.at[:, pl.ds(off, ECH)], crb, sem)

        def wait_idx(i, crb, sem):
            off = ebase + i * ECH
            pltpu.make_async_copy(ei_hbm.at[:, pl.ds(off, ECH)], crb, sem).wait()

        H = 24
        H2 = ECH - H

        def fire_gather(crb, buf, sem):
            pltpu.async_copy(x_hbm.at[crb.at[1, pl.ds(0, H)]],
                             buf.at[pl.ds(0, H)], sem)
            pltpu.async_copy(x_hbm.at[crb.at[1, pl.ds(H, H2)]],
                             buf.at[pl.ds(H, H2)], sem)

        def wait_gather(crb, buf, sem):
            pltpu.make_async_copy(x_hbm.at[crb.at[1, pl.ds(0, H)]],
                                  buf.at[pl.ds(0, H)], sem).wait()
            pltpu.make_async_copy(x_hbm.at[crb.at[1, pl.ds(H, H2)]],
                                  buf.at[pl.ds(H, H2)], sem).wait()

        for b in range(NBUF):
            fire_idx(b, crA[b], iA[b])
            fire_idx(b + NBUF, crB[b], iB[b])
        pltpu.sync_copy(z128_hbm, z128b)
        pltpu.sync_copy(z16_hbm, z16b)
        pltpu.sync_copy(ones_hbm, onesb)
        for b in range(NBUF):
            wait_idx(b, crA[b], iA[b])
            fire_gather(crA[b], bufs[b], gsem[b])

        def zero_body(k, _):
            c = sid + k * NS

            @pl.when(c < NRCH)
            def _():
                pltpu.sync_copy(z128b, acc_sh.at[pl.ds(c * RCH, RCH)])
                pltpu.sync_copy(z16b, dacc_sh.at[pl.ds(c * RCH, RCH)])
            return _

        lax.fori_loop(0, RK, zero_body, None)
        plsc.subcore_barrier()

        def half(base, crX, crY, isX, isY):
            for b in range(NBUF):
                wait_gather(crX[b], bufs[b], gsem[b])
                pltpu.async_copy(bufs[b], acc_sh.at[crX[b].at[0]], ssem[b],
                                 add=True)
                pltpu.async_copy(onesb, dacc_sh.at[crX[b].at[0]], ssem[b],
                                 add=True)
            for b in range(NBUF):
                i = base + b
                pltpu.make_async_copy(bufs[b], acc_sh.at[crX[b].at[0]],
                                      ssem[b]).wait()
                pltpu.make_async_copy(onesb, dacc_sh.at[crX[b].at[0]],
                                      ssem[b]).wait()

                @pl.when(i + 2 * NBUF < NECH)
                def _():
                    fire_idx(i + 2 * NBUF, crX[b], isX[b])

                @pl.when(i + NBUF < NECH)
                def _():
                    wait_idx(i + NBUF, crY[b], isY[b])
                    fire_gather(crY[b], bufs[b], gsem[b])

        def pair(gg, _):
            base0 = gg * 2 * NBUF
            half(base0, crA, crB, iA, iB)
            half(base0 + NBUF, crB, crA, iB, iA)
            return _

        lax.fori_loop(0, NGRP // 2, pair, None)
        plsc.subcore_barrier()

        def wb_body(k, _):
            c = sid + k * NS

            @pl.when(c < NRCH)
            def _():
                pltpu.sync_copy(acc_sh.at[pl.ds(c * RCH, RCH)], z128b)
                pltpu.sync_copy(z128b,
                                outv_hbm.at[pl.ds(cid * N + c * RCH, RCH)])
                pltpu.sync_copy(dacc_sh.at[pl.ds(c * RCH, RCH)], z16b)
                pltpu.sync_copy(z16b,
                                outd_hbm.at[pl.ds(cid * N + c * RCH, RCH),
                                            pl.ds(0, DW)])
            return _

        lax.fori_loop(0, RK, wb_body, None)

    return sc_kernel(x, ei, z128, z16, ones16)


def _mlp_body(x_ref, v0_ref, v1_ref, d0_ref, d1_ref,
              w1_ref, b1_ref, w2_ref, b2_ref,
              w3_ref, b3_ref, w4_ref, b4_ref, out_ref):
    def matt(a, w):
        return lax.dot_general(a, w, (((1,), (1,)), ((), ())),
                               preferred_element_type=jnp.float32)

    s = v0_ref[...] + v1_ref[...]
    deg = jnp.maximum(d0_ref[:, :1] + d1_ref[:, :1], 1.0)
    agg = s / deg
    h = jnp.concatenate([x_ref[...], agg], axis=1)
    h = jnp.maximum(matt(h, w1_ref[...]) + b1_ref[...], 0.0)
    h = jnp.maximum(matt(h, w2_ref[...]) + b2_ref[...], 0.0)
    h = jnp.maximum(matt(h, w3_ref[...]) + b3_ref[...], 0.0)
    out_ref[...] = matt(h, w4_ref[...]) + b4_ref[...]


def _tc_mlp(x, pv, pd, w1t, b1, w2t, b2, w3t, b3, w4t, b4):
    def wspec(shape):
        return pl.BlockSpec(shape, lambda i: (0, 0))

    return pl.pallas_call(
        _mlp_body,
        grid=(GRID,),
        in_specs=[
            pl.BlockSpec((BLK, D), lambda i: (i, 0)),
            pl.BlockSpec((BLK, D), lambda i: (i, 0)),
            pl.BlockSpec((BLK, D), lambda i: (i + GRID, 0)),
            pl.BlockSpec((BLK, D), lambda i: (i, 0)),
            pl.BlockSpec((BLK, D), lambda i: (i + GRID, 0)),
            wspec((HID, 2 * D)), wspec((1, HID)),
            wspec((HID, HID)), wspec((1, HID)),
            wspec((HID, HID)), wspec((1, HID)),
            wspec((D, HID)), wspec((1, D)),
        ],
        out_specs=pl.BlockSpec((BLK, D), lambda i: (i, 0)),
        out_shape=jax.ShapeDtypeStruct((N, D), jnp.float32),
    )(x, pv, pv, pd, pd, w1t, b1, w2t, b2, w3t, b3, w4t, b4)


def kernel(x, edge_index, W1, b1, W2, b2, W3, b3, W4, b4):
    ei = edge_index.astype(jnp.int32)
    z128 = jnp.zeros((RCH, D), jnp.float32)
    z16 = jnp.zeros((RCH, DW), jnp.float32)
    ones16 = jnp.ones((ECH, DW), jnp.float32)

    pv, pd = _sc_aggregate(x, ei, z128, z16, ones16)

    return _tc_mlp(
        x, pv, pd,
        W1, b1.reshape(1, HID),
        W2, b2.reshape(1, HID),
        W3, b3.reshape(1, HID),
        W4, b4.reshape(1, D),
    )


# speedup vs baseline: 1.1453x; 1.0020x over previous
import functools

import jax
import jax.numpy as jnp
from jax import lax
from jax.experimental import pallas as pl
from jax.experimental.pallas import tpu as pltpu
from jax.experimental.pallas import tpu_sc as plsc

N = 10000
D = 128
E = 320000
HID = 128
DW = 16
AUG = D + DW

NC = 2
NS = 16
NW = NC * NS
EPW = E // NW
ECH = 40
NECH = EPW // ECH
NBUF = 5
NGRP = NECH // NBUF
RCH = 40
NRCH = N // RCH
RK = (NRCH + NS - 1) // NS

BLK = 2000
GRID = N // BLK


def _sc_aggregate(x, ei, z128, z16, ones16):
    mesh = plsc.VectorSubcoreMesh(core_axis_name="c", subcore_axis_name="s")

    @functools.partial(
        pl.kernel,
        out_type=(jax.ShapeDtypeStruct((2 * N, D), jnp.float32),
                  jax.ShapeDtypeStruct((2 * N, D), jnp.float32)),
        mesh=mesh,
        compiler_params=pltpu.CompilerParams(use_tc_tiling_on_sc=False),
        scratch_types=[
            [pltpu.VMEM((2, ECH), jnp.int32) for _ in range(NBUF)],
            [pltpu.VMEM((2, ECH), jnp.int32) for _ in range(NBUF)],
            [pltpu.VMEM((ECH, D), jnp.float32) for _ in range(NBUF)],
            pltpu.VMEM((ECH, DW), jnp.float32),
            pltpu.VMEM((RCH, D), jnp.float32),
            pltpu.VMEM((RCH, DW), jnp.float32),
            pltpu.VMEM_SHARED((N, D), jnp.float32),
            pltpu.VMEM_SHARED((N, DW), jnp.float32),
            [pltpu.SemaphoreType.DMA for _ in range(NBUF)],
            [pltpu.SemaphoreType.DMA for _ in range(NBUF)],
            [pltpu.SemaphoreType.DMA for _ in range(NBUF)],
            [pltpu.SemaphoreType.DMA for _ in range(NBUF)],
        ],
    )
    def sc_kernel(x_hbm, ei_hbm, z128_hbm, z16_hbm, ones_hbm,
                  outv_hbm, outd_hbm,
                  crA, crB, bufs, onesb, z128b, z16b,
                  acc_sh, dacc_sh, iA, iB, gsem, ssem):
        cid = lax.axis_index("c")
        sid = lax.axis_index("s")
        wid = sid * NC + cid
        ebase = wid * EPW

        def fire_idx(i, crb, sem):
            off = ebase + i * ECH
            pltpu.async_copy(ei_hbm.at[:, pl.ds(off, ECH)], crb, sem)

        def wait_idx(i, crb, sem):
            off = ebase + i * ECH
            pltpu.make_async_copy(ei_hbm.at[:, pl.ds(off, ECH)], crb, sem).wait()

        def fire_gather(crb, buf, sem):
            pltpu.async_copy(x_hbm.at[crb.at[1]], buf, sem)

        def wait_gather(crb, buf, sem):
            pltpu.make_async_copy(x_hbm.at[crb.at[1]], buf, sem).wait()

        for b in range(NBUF):
            fire_idx(b, crA[b], iA[b])
            fire_idx(b + NBUF, crB[b], iB[b])
        pltpu.sync_copy(z128_hbm, z128b)
        pltpu.sync_copy(z16_hbm, z16b)
        pltpu.sync_copy(ones_hbm, onesb)
        for b in range(NBUF):
            wait_idx(b, crA[b], iA[b])
            fire_gather(crA[b], bufs[b], gsem[b])

        def zero_body(k, _):
            c = sid + k * NS

            @pl.when(c < NRCH)
            def _():
                pltpu.sync_copy(z128b, acc_sh.at[pl.ds(c * RCH, RCH)])
                pltpu.sync_copy(z16b, dacc_sh.at[pl.ds(c * RCH, RCH)])
            return _

        lax.fori_loop(0, RK, zero_body, None)
        plsc.subcore_barrier()

        def half(base, crX, crY, isX, isY):
            for b in range(NBUF):
                wait_gather(crX[b], bufs[b], gsem[b])
                pltpu.async_copy(bufs[b], acc_sh.at[crX[b].at[0]], ssem[b],
                                 add=True)
                pltpu.async_copy(onesb, dacc_sh.at[crX[b].at[0]], ssem[b],
                                 add=True)
            for b in range(NBUF):
                i = base + b
                pltpu.make_async_copy(bufs[b], acc_sh.at[crX[b].at[0]],
                                      ssem[b]).wait()
                pltpu.make_async_copy(onesb, dacc_sh.at[crX[b].at[0]],
                                      ssem[b]).wait()

                @pl.when(i + 2 * NBUF < NECH)
                def _():
                    fire_idx(i + 2 * NBUF, crX[b], isX[b])

                @pl.when(i + NBUF < NECH)
                def _():
                    wait_idx(i + NBUF, crY[b], isY[b])
                    fire_gather(crY[b], bufs[b], gsem[b])

        def pair(gg, _):
            base0 = gg * 2 * NBUF
            half(base0, crA, crB, iA, iB)
            half(base0 + NBUF, crB, crA, iB, iA)
            return _

        lax.fori_loop(0, NGRP // 2, pair, None)
        plsc.subcore_barrier()

        def wb_body(k, _):
            c = sid + k * NS

            @pl.when(c < NRCH)
            def _():
                pltpu.sync_copy(acc_sh.at[pl.ds(c * RCH, RCH)], z128b)
                pltpu.sync_copy(z128b,
                                outv_hbm.at[pl.ds(cid * N + c * RCH, RCH)])
                pltpu.sync_copy(dacc_sh.at[pl.ds(c * RCH, RCH)], z16b)
                pltpu.sync_copy(z16b,
                                outd_hbm.at[pl.ds(cid * N + c * RCH, RCH),
                                            pl.ds(0, DW)])
            return _

        lax.fori_loop(0, RK, wb_body, None)

    return sc_kernel(x, ei, z128, z16, ones16)


def _mlp_body(x_ref, v0_ref, v1_ref, d0_ref, d1_ref,
              w1_ref, b1_ref, w2_ref, b2_ref,
              w3_ref, b3_ref, w4_ref, b4_ref, out_ref):
    def matt(a, w):
        return lax.dot_general(a, w, (((1,), (1,)), ((), ())),
                               preferred_element_type=jnp.float32)

    s = v0_ref[...] + v1_ref[...]
    deg = jnp.maximum(d0_ref[:, :1] + d1_ref[:, :1], 1.0)
    agg = s / deg
    h = jnp.concatenate([x_ref[...], agg], axis=1)
    h = jnp.maximum(matt(h, w1_ref[...]) + b1_ref[...], 0.0)
    h = jnp.maximum(matt(h, w2_ref[...]) + b2_ref[...], 0.0)
    h = jnp.maximum(matt(h, w3_ref[...]) + b3_ref[...], 0.0)
    out_ref[...] = matt(h, w4_ref[...]) + b4_ref[...]


def _tc_mlp(x, pv, pd, w1t, b1, w2t, b2, w3t, b3, w4t, b4):
    def wspec(shape):
        return pl.BlockSpec(shape, lambda i: (0, 0))

    return pl.pallas_call(
        _mlp_body,
        grid=(GRID,),
        in_specs=[
            pl.BlockSpec((BLK, D), lambda i: (i, 0)),
            pl.BlockSpec((BLK, D), lambda i: (i, 0)),
            pl.BlockSpec((BLK, D), lambda i: (i + GRID, 0)),
            pl.BlockSpec((BLK, D), lambda i: (i, 0)),
            pl.BlockSpec((BLK, D), lambda i: (i + GRID, 0)),
            wspec((HID, 2 * D)), wspec((1, HID)),
            wspec((HID, HID)), wspec((1, HID)),
            wspec((HID, HID)), wspec((1, HID)),
            wspec((D, HID)), wspec((1, D)),
        ],
        out_specs=pl.BlockSpec((BLK, D), lambda i: (i, 0)),
        out_shape=jax.ShapeDtypeStruct((N, D), jnp.float32),
    )(x, pv, pv, pd, pd, w1t, b1, w2t, b2, w3t, b3, w4t, b4)


def kernel(x, edge_index, W1, b1, W2, b2, W3, b3, W4, b4):
    ei = edge_index.astype(jnp.int32)
    z128 = jnp.zeros((RCH, D), jnp.float32)
    z16 = jnp.zeros((RCH, DW), jnp.float32)
    ones16 = jnp.ones((ECH, DW), jnp.float32)

    pv, pd = _sc_aggregate(x, ei, z128, z16, ones16)

    return _tc_mlp(
        x, pv, pd,
        W1, b1.reshape(1, HID),
        W2, b2.reshape(1, HID),
        W3, b3.reshape(1, HID),
        W4, b4.reshape(1, D),
    )
